# baseline (speedup 1.0000x reference)
import functools

import jax
import jax.numpy as jnp
from jax import lax
from jax.experimental import pallas as pl
from jax.experimental.pallas import tpu as pltpu
from jax.experimental.pallas import tpu_sc as plsc

_DIM = 32
_NC = 2
_NS = 16
_NW = _NC * _NS
_CHUNK = 128
_L = 16
_BU = 4


@functools.lru_cache
def _build_repack(V):
    n_units = V // _CHUNK
    tail = V - n_units * _CHUNK
    nb = n_units // _BU
    assert nb * _BU == n_units
    k_hi = -(-nb // _NW)
    if k_hi % 2:
        k_hi += 1
    cols = _BU * _CHUNK

    @functools.partial(
        pl.kernel,
        mesh=plsc.VectorSubcoreMesh(
            core_axis_name="c", subcore_axis_name="s",
            num_cores=_NC, num_subcores=_NS),
        out_type=jax.ShapeDtypeStruct((V // 4, 128), jnp.float32),
        scratch_types=[
            pltpu.VMEM((_DIM, cols + 1), jnp.float32),
            pltpu.VMEM((_DIM, cols + 1), jnp.float32),
            pltpu.VMEM((_BU * _DIM, 128), jnp.float32),
            pltpu.VMEM((_BU * _DIM, 128), jnp.float32),
            pltpu.VMEM((16, 128), jnp.float32),
            pltpu.SemaphoreType.DMA,
            pltpu.SemaphoreType.DMA,
            pltpu.SemaphoreType.DMA,
            pltpu.SemaphoreType.DMA,
        ],
        compiler_params=pltpu.CompilerParams(
            use_tc_tiling_on_sc=True, needs_layout_passes=False),
    )
    def repack(w_hbm, wtail_hbm, out_hbm, bb0, bb1, tt0, tt1, tl,
               bsem0, bsem1, tsem0, tsem1):
        bb = (bb0, bb1)
        tt = (tt0, tt1)
        bsem = (bsem0, bsem1)
        tsem = (tsem0, tsem1)
        wid = lax.axis_index("s") * _NC + lax.axis_index("c")
        clo = lax.iota(jnp.int32, _L)

        def m_of(kk):
            return lax.min(kk * _NW + wid, nb - 1)

        def cpin(p, mm, g):
            return pltpu.make_async_copy(
                w_hbm.at[g, :, pl.ds(mm * cols, cols)],
                bb[p].at[pl.ds(g * 8, 8), pl.ds(0, cols)], bsem[p])

        def fire_in(p, mm):
            for g in range(4):
                cpin(p, mm, g).start()

        def wait_in(p, mm):
            for g in range(4):
                cpin(p, mm, g).wait()

        def cpout(p, mm):
            return pltpu.make_async_copy(
                tt[p], out_hbm.at[pl.ds(mm * _BU * _DIM, _BU * _DIM)],
                tsem[p])

        def shuffle(p):
            b, t = bb[p], tt[p]
            for u in range(_BU):

                @pl.loop(0, _DIM)
                def _(qq):
                    base = u * _CHUNK + qq * 4
                    for j in range(4):
                        colv = jnp.full((_L,), base + j, jnp.int32)
                        for c0 in (0, _L):
                            v = plsc.load_gather(b, [clo + c0, colv])
                            t[u * _DIM + qq, pl.ds(j * _DIM + c0, _L)] = v

        def step(kk, p, fire_next, wait_prev_out):
            if fire_next:
                fire_in(1 - p, m_of(kk + 1))
            wait_in(p, m_of(kk))
            if wait_prev_out:
                cpout(p, m_of(kk - 2)).wait()
            shuffle(p)
            cpout(p, m_of(kk)).start()

        fire_in(0, m_of(0))
        step(0, 0, True, False)
        step(1, 1, True, False)

        @pl.loop(2, k_hi - 2, step=2)
        def _(k):
            for p in range(2):
                step(k + p, p, True, True)

        step(k_hi - 2, 0, True, True)
        step(k_hi - 1, 1, False, True)
        cpout(0, m_of(k_hi - 2)).wait()
        cpout(1, m_of(k_hi - 1)).wait()

        if tail:
            @pl.when(wid == 1)
            def _():
                nq = tail // 4
                pltpu.sync_copy(wtail_hbm, tl.at[pl.ds(0, nq)])
                pltpu.sync_copy(tl.at[pl.ds(0, nq)],
                                out_hbm.at[pl.ds(n_units * _DIM, nq)])

    return repack


@functools.lru_cache
def _build_lookup(S, B, V):
    assert B == _NW * _CHUNK and S % 2 == 0

    @functools.partial(
        pl.kernel,
        mesh=plsc.VectorSubcoreMesh(
            core_axis_name="c", subcore_axis_name="s",
            num_cores=_NC, num_subcores=_NS),
        out_type=jax.ShapeDtypeStruct((S, _DIM, B), jnp.float32),
        scratch_types=[
            pltpu.VMEM((8, _CHUNK), jnp.int32),
            pltpu.VMEM((8, _CHUNK), jnp.int32),
            pltpu.VMEM((_CHUNK,), jnp.int32),
            pltpu.VMEM((_CHUNK,), jnp.int32),
            pltpu.VMEM((_CHUNK,), jnp.int32),
            pltpu.VMEM((_CHUNK,), jnp.int32),
            pltpu.VMEM((_CHUNK, 129), jnp.float32),
            pltpu.VMEM((_CHUNK, 129), jnp.float32),
            pltpu.VMEM((_DIM, _CHUNK), jnp.float32),
            pltpu.VMEM((_DIM, _CHUNK), jnp.float32),
            pltpu.SemaphoreType.DMA,
            pltpu.SemaphoreType.DMA,
            pltpu.SemaphoreType.DMA,
            pltpu.SemaphoreType.DMA,
            pltpu.SemaphoreType.DMA,
            pltpu.SemaphoreType.DMA,
        ],
        compiler_params=pltpu.CompilerParams(
            use_tc_tiling_on_sc=True, needs_layout_passes=False),
    )
    def lookup(idx_hbm, table_hbm, out_hbm, i0, i1, q0, q1, vo0, vo1,
               g0, g1, o0, o1, is0, is1, gs0, gs1, ws0, ws1):
        ibuf = (i0, i1)
        qbuf = (q0, q1)
        vbuf = (vo0, vo1)
        gbuf = (g0, g1)
        obuf = (o0, o1)
        isem = (is0, is1)
        gsem = (gs0, gs1)
        wsem = (ws0, ws1)
        wid = lax.axis_index("s") * _NC + lax.axis_index("c")
        blk = lax.div(wid, 8)
        sub = lax.rem(wid, 8)
        viota = lax.iota(jnp.int32, _L)

        def cpi(s, b):
            return pltpu.make_async_copy(idx_hbm.at[s, blk], ibuf[b], isem[b])

        def comp(b):
            for gi in range(_CHUNK // _L):
                v = ibuf[b][sub, pl.ds(gi * _L, _L)]
                qbuf[b][pl.ds(gi * _L, _L)] = lax.shift_right_logical(v, 2)
                vbuf[b][pl.ds(gi * _L, _L)] = lax.shift_left(v & 3, 5)

        def cpg(b):
            return pltpu.make_async_copy(
                table_hbm.at[qbuf[b]], gbuf[b].at[:, pl.ds(0, 128)], gsem[b])

        def cpw(s, b):
            return pltpu.make_async_copy(
                obuf[b], out_hbm.at[s, :, pl.ds(wid * _CHUNK, _CHUNK)],
                wsem[b])

        def extract(b):
            g, o = gbuf[b], obuf[b]

            @pl.loop(0, _CHUNK // _L)
            def _(gi):
                i0v = gi * _L
                voff = vbuf[b][pl.ds(i0v, _L)]
                rowv = viota + i0v
                for c in range(_DIM):
                    o[c, pl.ds(i0v, _L)] = plsc.load_gather(g, [rowv, voff + c])

        def step(s, b, fire_idx, wait_write):
            if fire_idx:
                cpi(s + 2, b).start()
            cpg(b).wait()
            if wait_write:
                cpw(s - 2, b).wait()
            extract(b)
            cpw(s, b).start()
            if fire_idx:
                cpi(s + 2, b).wait()
                comp(b)
                cpg(b).start()

        for b in range(2):
            cpi(b, b).start()
            cpi(b, b).wait()
            comp(b)
            cpg(b).start()
        step(0, 0, True, False)
        step(1, 1, True, False)

        @pl.loop(2, S - 2, step=2)
        def _(s):
            for b in range(2):
                step(s + b, b, True, True)

        step(S - 2, 0, False, True)
        step(S - 1, 1, False, True)
        cpw(S - 2, 0).wait()
        cpw(S - 1, 1).wait()

    return lookup


def kernel(x, weight):
    orig_shape = x.shape
    v, dim = weight.shape
    s = x.shape[-1]
    b = x.size // s
    wt4 = weight.T.reshape(4, 8, v)
    n_units = v // _CHUNK
    wtail = weight[n_units * _CHUNK:].reshape((v - n_units * _CHUNK) // 4, 128)
    xt = x.T.astype(jnp.int32).reshape(s, b // _CHUNK // 8, 8, _CHUNK)
    table = _build_repack(v)(wt4, wtail)
    out_phys = _build_lookup(s, b, v)(xt, table)
    return jnp.transpose(out_phys, (2, 0, 1)).reshape(orig_shape + (dim,))


# speedup vs baseline: 1.4305x; 1.4305x over previous
import functools

import jax
import jax.numpy as jnp
from jax import lax
from jax.experimental import pallas as pl
from jax.experimental.pallas import tpu as pltpu
from jax.experimental.pallas import tpu_sc as plsc

_DIM = 32
_NC = 2
_NS = 16
_NW = _NC * _NS
_CHUNK = 128
_L = 16
_BU = 4


@functools.lru_cache
def _build_repack(V):
    n_units = V // _CHUNK
    tail = V - n_units * _CHUNK
    nb = n_units // _BU
    assert nb * _BU == n_units
    k_hi = -(-nb // _NW)
    if k_hi % 2:
        k_hi += 1
    cols = _BU * _CHUNK

    @functools.partial(
        pl.kernel,
        mesh=plsc.VectorSubcoreMesh(
            core_axis_name="c", subcore_axis_name="s",
            num_cores=_NC, num_subcores=_NS),
        out_type=jax.ShapeDtypeStruct((V // 4, 128), jnp.float32),
        scratch_types=[
            pltpu.VMEM((_DIM, cols + 1), jnp.float32),
            pltpu.VMEM((_DIM, cols + 1), jnp.float32),
            pltpu.VMEM((_BU * _DIM, 128), jnp.float32),
            pltpu.VMEM((_BU * _DIM, 128), jnp.float32),
            pltpu.VMEM((16, 128), jnp.float32),
            pltpu.SemaphoreType.DMA,
            pltpu.SemaphoreType.DMA,
            pltpu.SemaphoreType.DMA,
            pltpu.SemaphoreType.DMA,
        ],
        compiler_params=pltpu.CompilerParams(
            use_tc_tiling_on_sc=True, needs_layout_passes=False),
    )
    def repack(w_hbm, wtail_hbm, out_hbm, bb0, bb1, tt0, tt1, tl,
               bsem0, bsem1, tsem0, tsem1):
        bb = (bb0, bb1)
        tt = (tt0, tt1)
        bsem = (bsem0, bsem1)
        tsem = (tsem0, tsem1)
        wid = lax.axis_index("s") * _NC + lax.axis_index("c")
        clo = lax.iota(jnp.int32, _L)

        def m_of(kk):
            return lax.min(kk * _NW + wid, nb - 1)

        def cpin(p, mm, g):
            return pltpu.make_async_copy(
                w_hbm.at[g, :, pl.ds(mm * cols, cols)],
                bb[p].at[pl.ds(g * 8, 8), pl.ds(0, cols)], bsem[p])

        def fire_in(p, mm):
            for g in range(4):
                cpin(p, mm, g).start()

        def wait_in(p, mm):
            for g in range(4):
                cpin(p, mm, g).wait()

        def cpout(p, mm):
            return pltpu.make_async_copy(
                tt[p], out_hbm.at[pl.ds(mm * _BU * _DIM, _BU * _DIM)],
                tsem[p])

        def shuffle(p):
            b, t = bb[p], tt[p]
            for u in range(_BU):

                @pl.loop(0, _DIM)
                def _(qq):
                    base = u * _CHUNK + qq * 4
                    for j in range(4):
                        colv = jnp.full((_L,), base + j, jnp.int32)
                        for c0 in (0, _L):
                            v = plsc.load_gather(b, [clo + c0, colv])
                            t[u * _DIM + qq, pl.ds(j * _DIM + c0, _L)] = v

        def step(kk, p, fire_next, wait_prev_out):
            if fire_next:
                fire_in(1 - p, m_of(kk + 1))
            wait_in(p, m_of(kk))
            if wait_prev_out:
                cpout(p, m_of(kk - 2)).wait()
            shuffle(p)
            cpout(p, m_of(kk)).start()

        fire_in(0, m_of(0))
        step(0, 0, True, False)
        step(1, 1, True, False)

        @pl.loop(2, k_hi - 2, step=2)
        def _(k):
            for p in range(2):
                step(k + p, p, True, True)

        step(k_hi - 2, 0, True, True)
        step(k_hi - 1, 1, False, True)
        cpout(0, m_of(k_hi - 2)).wait()
        cpout(1, m_of(k_hi - 1)).wait()

        if tail:
            @pl.when(wid == 1)
            def _():
                nq = tail // 4
                pltpu.sync_copy(wtail_hbm, tl.at[pl.ds(0, nq)])
                pltpu.sync_copy(tl.at[pl.ds(0, nq)],
                                out_hbm.at[pl.ds(n_units * _DIM, nq)])

    return repack


@functools.lru_cache
def _build_lookup(S, B, V):
    assert B == _NW * _CHUNK and S % 2 == 0

    @functools.partial(
        pl.kernel,
        mesh=plsc.VectorSubcoreMesh(
            core_axis_name="c", subcore_axis_name="s",
            num_cores=_NC, num_subcores=_NS),
        out_type=jax.ShapeDtypeStruct((S, _DIM, B), jnp.float32),
        scratch_types=[
            pltpu.VMEM((8, _CHUNK), jnp.int32),
            pltpu.VMEM((8, _CHUNK), jnp.int32),
            pltpu.VMEM((_CHUNK,), jnp.int32),
            pltpu.VMEM((_CHUNK,), jnp.int32),
            pltpu.VMEM((_CHUNK,), jnp.int32),
            pltpu.VMEM((_CHUNK,), jnp.int32),
            pltpu.VMEM((_CHUNK, 129), jnp.float32),
            pltpu.VMEM((_CHUNK, 129), jnp.float32),
            pltpu.VMEM((_DIM, _CHUNK), jnp.float32),
            pltpu.VMEM((_DIM, _CHUNK), jnp.float32),
            pltpu.SemaphoreType.DMA,
            pltpu.SemaphoreType.DMA,
            pltpu.SemaphoreType.DMA,
            pltpu.SemaphoreType.DMA,
            pltpu.SemaphoreType.DMA,
            pltpu.SemaphoreType.DMA,
        ],
        compiler_params=pltpu.CompilerParams(
            use_tc_tiling_on_sc=True, needs_layout_passes=False),
    )
    def lookup(idx_hbm, table_hbm, out_hbm, i0, i1, q0, q1, vo0, vo1,
               g0, g1, o0, o1, is0, is1, gs0, gs1, ws0, ws1):
        ibuf = (i0, i1)
        qbuf = (q0, q1)
        vbuf = (vo0, vo1)
        gbuf = (g0, g1)
        obuf = (o0, o1)
        isem = (is0, is1)
        gsem = (gs0, gs1)
        wsem = (ws0, ws1)
        wid = lax.axis_index("s") * _NC + lax.axis_index("c")
        blk = lax.div(wid, 8)
        sub = lax.rem(wid, 8)
        viota = lax.iota(jnp.int32, _L)

        def cpi(s, b):
            return pltpu.make_async_copy(idx_hbm.at[s, blk], ibuf[b], isem[b])

        def comp(b):
            for gi in range(_CHUNK // _L):
                v = ibuf[b][sub, pl.ds(gi * _L, _L)]
                qbuf[b][pl.ds(gi * _L, _L)] = lax.shift_right_logical(v, 2)
                vbuf[b][pl.ds(gi * _L, _L)] = lax.shift_left(v & 3, 5)

        def cpg(b):
            return pltpu.make_async_copy(
                table_hbm.at[qbuf[b]], gbuf[b].at[:, pl.ds(0, 128)], gsem[b])

        def cpw(s, b):
            return pltpu.make_async_copy(
                obuf[b], out_hbm.at[s, :, pl.ds(wid * _CHUNK, _CHUNK)],
                wsem[b])

        def extract(b):
            g, o = gbuf[b], obuf[b]

            @pl.loop(0, _CHUNK // _L)
            def _(gi):
                i0v = gi * _L
                voff = vbuf[b][pl.ds(i0v, _L)]
                rowv = viota + i0v
                for c in range(_DIM):
                    o[c, pl.ds(i0v, _L)] = plsc.load_gather(g, [rowv, voff + c])

        def step(s, b, fire_idx, wait_write):
            if fire_idx:
                cpi(s + 2, b).start()
            cpg(b).wait()
            if wait_write:
                cpw(s - 2, b).wait()
            extract(b)
            cpw(s, b).start()
            if fire_idx:
                cpi(s + 2, b).wait()
                comp(b)
                cpg(b).start()

        for b in range(2):
            cpi(b, b).start()
            cpi(b, b).wait()
            comp(b)
            cpg(b).start()
        step(0, 0, True, False)
        step(1, 1, True, False)

        @pl.loop(2, S - 2, step=2)
        def _(s):
            for b in range(2):
                step(s + b, b, True, True)

        step(S - 2, 0, False, True)
        step(S - 1, 1, False, True)
        cpw(S - 2, 0).wait()
        cpw(S - 1, 1).wait()

    return lookup


def kernel(x, weight):
    orig_shape = x.shape
    v, dim = weight.shape
    s = x.shape[-1]
    b = x.size // s
    xt = x.T.astype(jnp.int32).reshape(s, b // _CHUNK // 8, 8, _CHUNK)
    table = weight.reshape(v // 4, 128)
    out_phys = _build_lookup(s, b, v)(xt, table)
    return jnp.transpose(out_phys, (2, 0, 1)).reshape(orig_shape + (dim,))


# speedup vs baseline: 1.4310x; 1.0004x over previous
import functools

import jax
import jax.numpy as jnp
from jax import lax
from jax.experimental import pallas as pl
from jax.experimental.pallas import tpu as pltpu
from jax.experimental.pallas import tpu_sc as plsc

_DIM = 32
_NC = 2
_NS = 16
_NW = _NC * _NS
_CHUNK = 128
_L = 16


@functools.lru_cache
def _build_lookup(S, B, V):
    assert B == _NW * _CHUNK and S % 2 == 0

    @functools.partial(
        pl.kernel,
        mesh=plsc.VectorSubcoreMesh(
            core_axis_name="c", subcore_axis_name="s",
            num_cores=_NC, num_subcores=_NS),
        out_type=jax.ShapeDtypeStruct((S, _DIM, B), jnp.float32),
        scratch_types=[
            pltpu.VMEM((8, _CHUNK), jnp.int32),
            pltpu.VMEM((8, _CHUNK), jnp.int32),
            pltpu.VMEM((_CHUNK,), jnp.int32),
            pltpu.VMEM((_CHUNK,), jnp.int32),
            pltpu.VMEM((_CHUNK,), jnp.int32),
            pltpu.VMEM((_CHUNK,), jnp.int32),
            pltpu.VMEM((_CHUNK, 129), jnp.float32),
            pltpu.VMEM((_CHUNK, 129), jnp.float32),
            pltpu.VMEM((_DIM, _CHUNK), jnp.float32),
            pltpu.VMEM((_DIM, _CHUNK), jnp.float32),
            pltpu.SemaphoreType.DMA,
            pltpu.SemaphoreType.DMA,
            pltpu.SemaphoreType.DMA,
            pltpu.SemaphoreType.DMA,
            pltpu.SemaphoreType.DMA,
            pltpu.SemaphoreType.DMA,
        ],
        compiler_params=pltpu.CompilerParams(
            use_tc_tiling_on_sc=True, needs_layout_passes=False),
    )
    def lookup(idx_hbm, table_hbm, out_hbm, i0, i1, q0, q1, vo0, vo1,
               g0, g1, o0, o1, is0, is1, gs0, gs1, ws0, ws1):
        ibuf = (i0, i1)
        qbuf = (q0, q1)
        vbuf = (vo0, vo1)
        gbuf = (g0, g1)
        obuf = (o0, o1)
        isem = (is0, is1)
        gsem = (gs0, gs1)
        wsem = (ws0, ws1)
        wid = lax.axis_index("s") * _NC + lax.axis_index("c")
        blk = lax.div(wid, 8)
        sub = lax.rem(wid, 8)
        viota = lax.iota(jnp.int32, _L)

        def cpi(s, b):
            return pltpu.make_async_copy(idx_hbm.at[s, blk], ibuf[b], isem[b])

        def comp(b):
            for gi in range(_CHUNK // _L):
                v = ibuf[b][sub, pl.ds(gi * _L, _L)]
                qbuf[b][pl.ds(gi * _L, _L)] = lax.shift_right_logical(v, 2)
                vbuf[b][pl.ds(gi * _L, _L)] = lax.shift_left(v & 3, 5)

        def cpg(b):
            return pltpu.make_async_copy(
                table_hbm.at[qbuf[b]], gbuf[b].at[:, pl.ds(0, 128)], gsem[b])

        def cpw(s, b):
            return pltpu.make_async_copy(
                obuf[b], out_hbm.at[s, :, pl.ds(wid * _CHUNK, _CHUNK)],
                wsem[b])

        def extract(b):
            g, o = gbuf[b], obuf[b]

            @pl.loop(0, _CHUNK // _L)
            def _(gi):
                i0v = gi * _L
                voff = vbuf[b][pl.ds(i0v, _L)]
                rowv = viota + i0v
                for c in range(_DIM):
                    o[c, pl.ds(i0v, _L)] = plsc.load_gather(g, [rowv, voff + c])

        def step(s, b, fire_idx, wait_write):
            if fire_idx:
                cpi(s + 2, b).start()
            cpg(b).wait()
            if wait_write:
                cpw(s - 2, b).wait()
            extract(b)
            cpw(s, b).start()
            if fire_idx:
                cpi(s + 2, b).wait()
                comp(b)
                cpg(b).start()

        for b in range(2):
            cpi(b, b).start()
            cpi(b, b).wait()
            comp(b)
            cpg(b).start()
        step(0, 0, True, False)
        step(1, 1, True, False)

        @pl.loop(2, S - 2, step=2)
        def _(s):
            for b in range(2):
                step(s + b, b, True, True)

        step(S - 2, 0, False, True)
        step(S - 1, 1, False, True)
        cpw(S - 2, 0).wait()
        cpw(S - 1, 1).wait()

    return lookup


def kernel(x, weight):
    orig_shape = x.shape
    v, dim = weight.shape
    s = x.shape[-1]
    b = x.size // s
    xt = x.T.astype(jnp.int32).reshape(s, b // _CHUNK // 8, 8, _CHUNK)
    table = weight.reshape(v // 4, 128)
    out_phys = _build_lookup(s, b, v)(xt, table)
    return jnp.transpose(out_phys, (2, 0, 1)).reshape(orig_shape + (dim,))
